# baseline (speedup 1.0000x reference)
import functools

import jax
import jax.numpy as jnp
from jax import lax
from jax.experimental import pallas as pl
from jax.experimental.pallas import tpu as pltpu
from jax.experimental.pallas import tpu_sc as plsc

VOCAB = 100000
EMB = 128
NLAB = 1000
B = 4096
L = 200

NC, NS, LANES = 2, 16, 16
NW = NC * NS
BPW = B // NW
IDS_PW = BPW * L
NV = EMB // LANES
GSEG = 40
NSEG = L // GSEG
UNROLL = 8
NBUF = 4
NRING = 4


def _sc_sum_body(ids_hbm, emb_hbm, out_hbm, idx_v, rows_v, ring_v,
                 sem0, sem1, sem2, sem3, semo):
    c = lax.axis_index("c")
    s = lax.axis_index("s")
    wid = s * NC + c
    base = wid * BPW
    pltpu.sync_copy(ids_hbm.at[pl.ds(base * L, IDS_PW)], idx_v)
    sems = (sem0, sem1, sem2, sem3)

    def fire(r, buf):
        off = r * L
        for q in range(NSEG):
            pltpu.make_async_copy(
                emb_hbm.at[idx_v.at[pl.ds(off + q * GSEG, GSEG)]],
                rows_v.at[buf, pl.ds(q * GSEG, GSEG)], sems[buf]).start()

    def wait(buf):
        for q in range(NSEG):
            pltpu.make_async_copy(
                emb_hbm.at[idx_v.at[pl.ds(0, GSEG)]],
                rows_v.at[buf, pl.ds(q * GSEG, GSEG)], sems[buf]).wait()

    zeros = tuple(jnp.zeros((LANES,), jnp.float32) for _ in range(NV))

    def accum(buf, m):
        def acc_body(t, acc):
            j = t * UNROLL
            for u in range(UNROLL):
                acc = tuple(
                    acc[k] + rows_v[buf, j + u, pl.ds(k * LANES, LANES)]
                    for k in range(NV))
            return acc

        acc = lax.fori_loop(0, L // UNROLL, acc_body, zeros)
        for k in range(NV):
            ring_v[m, pl.ds(k * LANES, LANES)] = acc[k]

    def out_fire(r, m):
        pltpu.make_async_copy(
            ring_v.at[pl.ds(m, 1)], out_hbm.at[pl.ds(base + r, 1)],
            semo).start()

    def out_drain(m):
        pltpu.make_async_copy(
            ring_v.at[pl.ds(m, 1)], out_hbm.at[pl.ds(base, 1)], semo).wait()

    for buf in range(NBUF):
        fire(buf, buf)

    def quad_body(g, carry):
        r0 = NBUF * g
        for buf in range(NBUF):
            r = r0 + buf
            m = r % NRING

            @pl.when(r >= NRING)
            def _(m=m):
                out_drain(m)

            wait(buf)
            accum(buf, m)
            out_fire(r, m)

            @pl.when(r + NBUF < BPW)
            def _(buf=buf, r=r):
                fire(r + NBUF, buf)
        return carry

    lax.fori_loop(0, BPW // NBUF, quad_body, 0)
    for _ in range(NRING):
        out_drain(0)


_sc_sum = functools.partial(
    pl.kernel,
    out_type=jax.ShapeDtypeStruct((B, EMB), jnp.float32),
    mesh=plsc.VectorSubcoreMesh(core_axis_name="c", subcore_axis_name="s"),
    scratch_types=[
        pltpu.VMEM((IDS_PW,), jnp.int32),
        pltpu.VMEM((NBUF, L, EMB), jnp.float32),
        pltpu.VMEM((NRING, EMB), jnp.float32),
        pltpu.SemaphoreType.DMA,
        pltpu.SemaphoreType.DMA,
        pltpu.SemaphoreType.DMA,
        pltpu.SemaphoreType.DMA,
        pltpu.SemaphoreType.DMA,
    ],
)(_sc_sum_body)


def _tc_fc_body(sum_ref, len_ref, w_ref, b_ref, out_ref):
    inv = 1.0 / jnp.maximum(len_ref[...], 1.0)
    mean = sum_ref[...] * inv
    out_ref[...] = (
        jnp.dot(mean, w_ref[...], preferred_element_type=jnp.float32)
        + b_ref[...])


def kernel(ids, lengths, emb, W, b):
    ids_flat = ids.reshape(-1).astype(jnp.int32)
    summed = _sc_sum(ids_flat, emb)

    lenf = lengths.astype(jnp.float32).reshape(B, 1)
    bp = b.reshape(1, NLAB)

    BT = 4096
    out = pl.pallas_call(
        _tc_fc_body,
        grid=(B // BT,),
        in_specs=[
            pl.BlockSpec((BT, EMB), lambda i: (i, 0)),
            pl.BlockSpec((BT, 1), lambda i: (i, 0)),
            pl.BlockSpec((EMB, NLAB), lambda i: (0, 0)),
            pl.BlockSpec((1, NLAB), lambda i: (0, 0)),
        ],
        out_specs=pl.BlockSpec((BT, NLAB), lambda i: (i, 0)),
        out_shape=jax.ShapeDtypeStruct((B, NLAB), jnp.float32),
    )(summed, lenf, W, bp)
    return out


# speedup vs baseline: 1.0016x; 1.0016x over previous
import functools

import jax
import jax.numpy as jnp
from jax import lax
from jax.experimental import pallas as pl
from jax.experimental.pallas import tpu as pltpu
from jax.experimental.pallas import tpu_sc as plsc

VOCAB = 100000
EMB = 128
NLAB = 1000
B = 4096
L = 200

NC, NS, LANES = 2, 16, 16
NW = NC * NS
BPW = B // NW
IDS_PW = BPW * L
NV = EMB // LANES
GSEG = 40
NSEG = L // GSEG
UNROLL = 8
NBUF = 4
NRING = 4


def _sc_sum_body(ids_hbm, emb_hbm, out_hbm, idx_v, rows_v, ring_v,
                 sem0, sem1, sem2, sem3, semo):
    c = lax.axis_index("c")
    s = lax.axis_index("s")
    wid = s * NC + c
    base = wid * BPW
    pltpu.sync_copy(ids_hbm.at[pl.ds(base * L, IDS_PW)], idx_v)
    sems = (sem0, sem1, sem2, sem3)

    def fire(r, buf):
        off = r * L
        for q in range(NSEG):
            pltpu.make_async_copy(
                emb_hbm.at[idx_v.at[pl.ds(off + q * GSEG, GSEG)]],
                rows_v.at[buf, pl.ds(q * GSEG, GSEG)], sems[buf]).start()

    def wait(buf):
        for q in range(NSEG):
            pltpu.make_async_copy(
                emb_hbm.at[idx_v.at[pl.ds(0, GSEG)]],
                rows_v.at[buf, pl.ds(q * GSEG, GSEG)], sems[buf]).wait()

    zeros = tuple(jnp.zeros((LANES,), jnp.float32) for _ in range(NV))

    def accum(buf, m):
        def acc_body(t, acc):
            j = t * UNROLL
            for u in range(UNROLL):
                acc = tuple(
                    acc[k] + rows_v[buf, j + u, pl.ds(k * LANES, LANES)]
                    for k in range(NV))
            return acc

        acc = lax.fori_loop(0, L // UNROLL, acc_body, zeros)
        for k in range(NV):
            ring_v[m, pl.ds(k * LANES, LANES)] = acc[k]

    def out_fire(r, m):
        pltpu.make_async_copy(
            ring_v.at[pl.ds(m, 1)], out_hbm.at[pl.ds(base + r, 1)],
            semo).start()

    def out_drain(m):
        pltpu.make_async_copy(
            ring_v.at[pl.ds(m, 1)], out_hbm.at[pl.ds(base, 1)], semo).wait()

    for buf in range(NBUF):
        fire(buf, buf)

    def quad_body(g, carry):
        r0 = NBUF * g
        for buf in range(NBUF):
            r = r0 + buf
            m = r % NRING

            @pl.when(r >= NRING)
            def _(m=m):
                out_drain(m)

            wait(buf)
            accum(buf, m)
            out_fire(r, m)

            @pl.when(r + NBUF < BPW)
            def _(buf=buf, r=r):
                fire(r + NBUF, buf)
        return carry

    lax.fori_loop(0, BPW // NBUF, quad_body, 0)
    for _ in range(NRING):
        out_drain(0)


_sc_sum = functools.partial(
    pl.kernel,
    out_type=jax.ShapeDtypeStruct((B, EMB), jnp.float32),
    mesh=plsc.VectorSubcoreMesh(core_axis_name="c", subcore_axis_name="s"),
    scratch_types=[
        pltpu.VMEM((IDS_PW,), jnp.int32),
        pltpu.VMEM((NBUF, L, EMB), jnp.float32),
        pltpu.VMEM((NRING, EMB), jnp.float32),
        pltpu.SemaphoreType.DMA,
        pltpu.SemaphoreType.DMA,
        pltpu.SemaphoreType.DMA,
        pltpu.SemaphoreType.DMA,
        pltpu.SemaphoreType.DMA,
    ],
)(_sc_sum_body)


def _tc_fc_body(sum_ref, len_ref, w_ref, b_ref, out_ref):
    inv = 1.0 / jnp.maximum(len_ref[...], 1.0)
    mean = sum_ref[...] * inv
    out_ref[...] = (
        jnp.dot(mean, w_ref[...], preferred_element_type=jnp.float32)
        + b_ref[...])


def kernel(ids, lengths, emb, W, b):
    ids_flat = ids.reshape(-1).astype(jnp.int32)
    summed = _sc_sum(ids_flat, emb)

    lenf = lengths.astype(jnp.float32).reshape(B, 1)
    bp = b.reshape(1, NLAB)

    BT = 2048
    out = pl.pallas_call(
        _tc_fc_body,
        grid=(B // BT,),
        in_specs=[
            pl.BlockSpec((BT, EMB), lambda i: (i, 0)),
            pl.BlockSpec((BT, 1), lambda i: (i, 0)),
            pl.BlockSpec((EMB, NLAB), lambda i: (0, 0)),
            pl.BlockSpec((1, NLAB), lambda i: (0, 0)),
        ],
        out_specs=pl.BlockSpec((BT, NLAB), lambda i: (i, 0)),
        out_shape=jax.ShapeDtypeStruct((B, NLAB), jnp.float32),
    )(summed, lenf, W, bp)
    return out
